# Initial kernel scaffold; baseline (speedup 1.0000x reference)
import jax
import jax.numpy as jnp
from jax.experimental import pallas as pl


def kernel(x, edge_index, edge_weight, W1, b1, W2, b2, Wf, bf):
    raise NotImplementedError("write your pallas kernel here")


# speedup vs baseline: 13.4284x; 13.4284x over previous
import functools

import jax
import jax.numpy as jnp
from jax import lax
from jax.experimental import pallas as pl
from jax.experimental.pallas import tpu as pltpu
from jax.experimental.pallas import tpu_sc as plsc

N = 10000
E = 320000
D = 128
NC = 2
NS = 16
NW = NC * NS
EPW = E // NW
K = 80
NCH = EPW // K
RPT = 640
NPAD = NS * RPT
ZR = 128

def _zero_acc_rows(zbuf, acc, s):
    def zrow(i, _):
        for cg in range(8):
            zbuf[i, pl.ds(cg * 16, 16)] = jnp.zeros((16,), jnp.float32)
        return 0

    lax.fori_loop(0, K, zrow, 0)
    for blk in range(RPT // K):
        pltpu.sync_copy(zbuf, acc.at[pl.ds(s * RPT + blk * K, K)])


def _sc_degree_body(cols_hbm, ew_hbm, out_hbm, cols_v, ew_v, zb, dacc):
    c = lax.axis_index("c")
    s = lax.axis_index("s")
    wid = s * NC + c

    def zrow(i, _):
        zb[pl.ds(i * 16, 16)] = jnp.zeros((16,), jnp.float32)
        return 0

    lax.fori_loop(0, ZR // 16, zrow, 0)
    for blk in range(RPT // ZR):
        pltpu.sync_copy(zb, dacc.at[pl.ds(s * RPT + blk * ZR, ZR)])
    plsc.subcore_barrier()

    pltpu.sync_copy(cols_hbm.at[wid], cols_v)
    pltpu.sync_copy(ew_hbm.at[wid], ew_v)

    def chunk(j, _):
        pltpu.sync_copy(ew_v.at[j], dacc.at[cols_v.at[j]], add=True)
        return 0

    lax.fori_loop(0, NCH, chunk, 0)
    plsc.subcore_barrier()
    pltpu.sync_copy(dacc.at[pl.ds(s * RPT, RPT)], out_hbm.at[c, pl.ds(s * RPT, RPT)])


def _sc_aggregate_body(idx_hbm, ew_hbm, g_hbm, out_hbm,
                       idxc, ew_v, gbuf, acc, sem):
    c = lax.axis_index("c")
    s = lax.axis_index("s")
    wid = s * NC + c

    _zero_acc_rows(gbuf, acc, s)
    plsc.subcore_barrier()

    pltpu.sync_copy(ew_hbm.at[wid], ew_v)

    def chunk(j, _):
        pltpu.sync_copy(idx_hbm.at[wid, j], idxc)
        pltpu.async_copy(g_hbm.at[idxc.at[0]], gbuf, sem).wait()

        def blk16(b, _):
            ewv = ew_v[pl.ds(j * K + b * 16, 16)]
            for rr in range(16):
                s_ew = ewv[rr]
                r = b * 16 + rr
                for cg in range(8):
                    sl = pl.ds(cg * 16, 16)
                    gbuf[r, sl] = gbuf[r, sl] * s_ew
            return 0

        lax.fori_loop(0, K // 16, blk16, 0)
        pltpu.sync_copy(gbuf, acc.at[idxc.at[1]], add=True)
        return 0

    lax.fori_loop(0, NCH, chunk, 0)
    plsc.subcore_barrier()
    pltpu.sync_copy(acc.at[pl.ds(s * RPT, RPT)], out_hbm.at[c, pl.ds(s * RPT, RPT)])


@functools.lru_cache(maxsize=None)
def _build_sc_kernels():
    mesh = plsc.VectorSubcoreMesh(core_axis_name="c", subcore_axis_name="s",
                                  num_cores=NC, num_subcores=NS)
    sc_degree = pl.kernel(
        _sc_degree_body,
        out_type=jax.ShapeDtypeStruct((NC, NPAD), jnp.float32),
        mesh=mesh,
        scratch_types=[
            pltpu.VMEM((NCH, K), jnp.int32),
            pltpu.VMEM((NCH, K), jnp.float32),
            pltpu.VMEM((ZR,), jnp.float32),
            pltpu.VMEM_SHARED((NPAD,), jnp.float32),
        ],
    )
    sc_aggregate = pl.kernel(
        _sc_aggregate_body,
        out_type=jax.ShapeDtypeStruct((NC, NPAD, D), jnp.float32),
        mesh=mesh,
        scratch_types=[
            pltpu.VMEM((2, K), jnp.int32),
            pltpu.VMEM((EPW,), jnp.float32),
            pltpu.VMEM((K, D), jnp.float32),
            pltpu.VMEM_SHARED((NPAD, D), jnp.float32),
            pltpu.SemaphoreType.DMA,
        ],
    )
    return sc_degree, sc_aggregate


def _mm_scale_body(x_ref, w_ref, d_ref, o_ref):
    acc = lax.dot_general(x_ref[...], w_ref[...], (((1,), (1,)), ((), ())),
                          preferred_element_type=jnp.float32)
    o_ref[...] = acc * d_ref[...]


def _layer_body(s0_ref, s1_ref, g_ref, d_ref, b_ref, w_ref, o_ref):
    x2 = jnp.maximum((s0_ref[...] + s1_ref[...] + g_ref[...]) * d_ref[...]
                     + b_ref[...], 0.0)
    acc = lax.dot_general(x2, w_ref[...], (((1,), (1,)), ((), ())),
                          preferred_element_type=jnp.float32)
    o_ref[...] = acc * d_ref[...]


def _final_body(s0_ref, s1_ref, g_ref, d_ref, b_ref, w_ref, bf_ref, o_ref):
    h = jnp.maximum((s0_ref[...] + s1_ref[...] + g_ref[...]) * d_ref[...]
                    + b_ref[...], 0.0)
    acc = lax.dot_general(h, w_ref[...], (((1,), (1,)), ((), ())),
                          preferred_element_type=jnp.float32)
    o_ref[...] = jax.nn.sigmoid(acc + bf_ref[...])


_BLK = 1000
_GRID = N // _BLK


def _row_spec(d):
    return pl.BlockSpec((_BLK, d), lambda i: (i, 0))


def _full_spec(r, d):
    return pl.BlockSpec((r, d), lambda i: (0, 0))


def kernel(x, edge_index, edge_weight, W1, b1, W2, b2, Wf, bf):
    sc_degree, sc_aggregate = _build_sc_kernels()
    rows3 = edge_index[0].reshape(NW, NCH, K)
    cols3 = edge_index[1].reshape(NW, NCH, K)
    idx4 = jnp.stack([rows3, cols3], axis=2)
    ew3 = edge_weight.reshape(NW, NCH, K)
    ew2 = edge_weight.reshape(NW, EPW)

    degp = sc_degree(cols3, ew3)
    deg = degp[0, :N] + degp[1, :N] + 1.0
    dis = jnp.where(deg > 0, lax.rsqrt(jnp.maximum(deg, 1e-12)), 0.0)
    disb = jnp.broadcast_to(dis[:, None], (N, D))

    g1 = pl.pallas_call(
        _mm_scale_body,
        grid=(_GRID,),
        in_specs=[_row_spec(D), _full_spec(D, D), _row_spec(D)],
        out_specs=_row_spec(D),
        out_shape=jax.ShapeDtypeStruct((N, D), jnp.float32),
    )(x, W1, disb)

    S1 = sc_aggregate(idx4, ew2, g1)

    g2 = pl.pallas_call(
        _layer_body,
        grid=(_GRID,),
        in_specs=[_row_spec(D), _row_spec(D), _row_spec(D), _row_spec(D),
                  _full_spec(1, D), _full_spec(D, D)],
        out_specs=_row_spec(D),
        out_shape=jax.ShapeDtypeStruct((N, D), jnp.float32),
    )(S1[0, :N], S1[1, :N], g1, disb, b1.reshape(1, D), W2)

    S2 = sc_aggregate(idx4, ew2, g2)

    out = pl.pallas_call(
        _final_body,
        grid=(_GRID,),
        in_specs=[_row_spec(D), _row_spec(D), _row_spec(D), _row_spec(D),
                  _full_spec(1, D), _full_spec(16, D), _full_spec(1, 16)],
        out_specs=_row_spec(16),
        out_shape=jax.ShapeDtypeStruct((N, 16), jnp.float32),
    )(S2[0, :N], S2[1, :N], g2, disb, b2.reshape(1, D), Wf, bf.reshape(1, 16))

    return out
